# baseline (speedup 1.0000x reference)
import jax
import jax.numpy as jnp
from jax import lax
from jax.experimental import pallas as pl
from jax.experimental.pallas import tpu as pltpu
from jax.experimental.pallas import tpu_sc as plsc

NUM_USERS = 100000
VOCAB = 1000
BATCH = 4096
GAIN = 50.0
CLIP = 2000.0

_L = 16
_NW = 32
_BPW = BATCH // _NW
_C = 32
_NCHUNK = _BPW // _C
_NSLICE = VOCAB // _L


def _sc_body(uid_hbm, w_hbm, out_hbm, idx_v, in_buf, out_buf, sem):
    wid = lax.axis_index("s") * 2 + lax.axis_index("c")
    base = wid * _BPW
    pltpu.sync_copy(uid_hbm.at[pl.ds(base, _BPW)], idx_v)

    def row(r, carry):
        for j in range(_NSLICE):
            x = in_buf[r, pl.ds(j * _L, _L)]
            out_buf[r, pl.ds(j * _L, _L)] = jnp.clip(x * GAIN, -CLIP, CLIP)
        x = in_buf[r, pl.ds(VOCAB - _L, _L)]
        out_buf[r, pl.ds(VOCAB - _L, _L)] = jnp.clip(x * GAIN, -CLIP, CLIP)
        return carry

    for c in range(_NCHUNK):
        pltpu.async_copy(w_hbm.at[idx_v.at[pl.ds(c * _C, _C)]], in_buf,
                         sem).wait()
        lax.fori_loop(0, _C, row, 0)
        pltpu.sync_copy(out_buf, out_hbm.at[pl.ds(base + c * _C, _C)])


def kernel(user_ids, weight):
    mesh = plsc.VectorSubcoreMesh(core_axis_name="c", subcore_axis_name="s")
    f = pl.kernel(
        _sc_body,
        mesh=mesh,
        out_type=jax.ShapeDtypeStruct((BATCH, VOCAB), jnp.float32),
        scratch_types=[
            pltpu.VMEM((_BPW,), jnp.int32),
            pltpu.VMEM((_C, VOCAB), jnp.float32),
            pltpu.VMEM((_C, VOCAB), jnp.float32),
            pltpu.SemaphoreType.DMA,
        ],
        compiler_params=pltpu.CompilerParams(use_tc_tiling_on_sc=False),
    )
    return f(user_ids.astype(jnp.int32), weight)


# speedup vs baseline: 1.2227x; 1.2227x over previous
import jax
import jax.numpy as jnp
from jax import lax
from jax.experimental import pallas as pl
from jax.experimental.pallas import tpu as pltpu
from jax.experimental.pallas import tpu_sc as plsc

NUM_USERS = 100000
VOCAB = 1000
BATCH = 4096
GAIN = 50.0
CLIP = 2000.0

_L = 16
_NW = 32
_BPW = BATCH // _NW
_NB = _BPW // 8
_NSLICE = VOCAB // _L


def _sc_body(uid_hbm, w_hbm, out_hbm, idx_v, band_v, lane_v, in_buf,
             out_buf, sem):
    wid = lax.axis_index("s") * 2 + lax.axis_index("c")
    base = wid * _BPW
    pltpu.sync_copy(uid_hbm.at[pl.ds(base, _BPW)], idx_v)
    for j in range(_BPW // _L):
        ids = idx_v[pl.ds(j * _L, _L)]
        band_v[pl.ds(j * _L, _L)] = lax.shift_right_logical(ids, 3)
        lane_v[pl.ds(j * _L, _L)] = lax.bitwise_and(ids, 7)

    def do_band(ob, carry):
        off = pl.multiple_of(ob * 8, 8)
        copies = []
        for k in range(8):
            band = band_v[pl.ds(ob * 8 + k, _L)][0]
            copies.append(pltpu.async_copy(w_hbm.at[band], in_buf.at[k],
                                           sem))
        for c in copies:
            c.wait()

        def do_row(k, c2):
            lane = lane_v[pl.ds(ob * 8 + k, _L)][0]
            for j in range(_NSLICE):
                x = in_buf[k, lane, pl.ds(j * _L, _L)]
                out_buf[k, pl.ds(j * _L, _L)] = jnp.clip(x * GAIN, -CLIP,
                                                         CLIP)
            x = in_buf[k, lane, pl.ds(VOCAB - _L, _L)]
            out_buf[k, pl.ds(VOCAB - _L, _L)] = jnp.clip(x * GAIN, -CLIP,
                                                         CLIP)
            return c2

        lax.fori_loop(0, 8, do_row, 0)
        pltpu.sync_copy(out_buf,
                        out_hbm.at[pl.ds(pl.multiple_of(base + off, 8), 8)])
        return carry

    lax.fori_loop(0, _NB, do_band, 0)


def kernel(user_ids, weight):
    w3 = weight.reshape(NUM_USERS // 8, 8, VOCAB)
    mesh = plsc.VectorSubcoreMesh(core_axis_name="c", subcore_axis_name="s")
    f = pl.kernel(
        _sc_body,
        mesh=mesh,
        out_type=jax.ShapeDtypeStruct((BATCH, VOCAB), jnp.float32),
        scratch_types=[
            pltpu.VMEM((_BPW,), jnp.int32),
            pltpu.VMEM((_BPW + _L,), jnp.int32),
            pltpu.VMEM((_BPW + _L,), jnp.int32),
            pltpu.VMEM((8, 8, VOCAB), jnp.float32),
            pltpu.VMEM((8, VOCAB), jnp.float32),
            pltpu.SemaphoreType.DMA,
        ],
    )
    return f(user_ids.astype(jnp.int32), w3)


# speedup vs baseline: 4.4524x; 3.6414x over previous
import jax
import jax.numpy as jnp
from jax import lax
from jax.experimental import pallas as pl
from jax.experimental.pallas import tpu as pltpu
from jax.experimental.pallas import tpu_sc as plsc

NUM_USERS = 100000
VOCAB = 1000
BATCH = 4096
GAIN = 50.0
CLIP = 2000.0

_L = 16
_NW = 32
_BPW = BATCH // _NW
_NB = _BPW // 8
_NSLICE = VOCAB // _L


def _sc_body(uid_hbm, w_hbm, out_hbm, idx_v, band_v, lane_v, in_buf,
             out_buf, sem):
    wid = lax.axis_index("s") * 2 + lax.axis_index("c")
    base = wid * _BPW
    pltpu.sync_copy(uid_hbm.at[pl.ds(base, _BPW)], idx_v)
    for j in range(_BPW // _L):
        ids = idx_v[pl.ds(j * _L, _L)]
        band_v[pl.ds(j * _L, _L)] = lax.shift_right_logical(ids, 3)
        lane_v[pl.ds(j * _L, _L)] = lax.bitwise_and(ids, 7)

    def do_band(ob, carry):
        off = pl.multiple_of(ob * 8, 8)
        copies = []
        for k in range(8):
            row0 = pl.multiple_of(band_v[pl.ds(ob * 8 + k, _L)][0] * 8, 8)
            copies.append(pltpu.async_copy(w_hbm.at[pl.ds(row0, 8)],
                                           in_buf.at[k], sem))
        for c in copies:
            c.wait()

        def do_row(k, c2):
            lane = lane_v[pl.ds(ob * 8 + k, _L)][0]
            for j in range(_NSLICE):
                x = in_buf[k, lane, pl.ds(j * _L, _L)]
                out_buf[k, pl.ds(j * _L, _L)] = jnp.clip(x * GAIN, -CLIP,
                                                         CLIP)
            x = in_buf[k, lane, pl.ds(VOCAB - _L, _L)]
            out_buf[k, pl.ds(VOCAB - _L, _L)] = jnp.clip(x * GAIN, -CLIP,
                                                         CLIP)
            return c2

        lax.fori_loop(0, 8, do_row, 0)
        pltpu.sync_copy(out_buf,
                        out_hbm.at[pl.ds(pl.multiple_of(base + off, 8), 8)])
        return carry

    lax.fori_loop(0, _NB, do_band, 0)


def kernel(user_ids, weight):
    mesh = plsc.VectorSubcoreMesh(core_axis_name="c", subcore_axis_name="s")
    f = pl.kernel(
        _sc_body,
        mesh=mesh,
        out_type=jax.ShapeDtypeStruct((BATCH, VOCAB), jnp.float32),
        scratch_types=[
            pltpu.VMEM((_BPW,), jnp.int32),
            pltpu.VMEM((_BPW + _L,), jnp.int32),
            pltpu.VMEM((_BPW + _L,), jnp.int32),
            pltpu.VMEM((8, 8, VOCAB), jnp.float32),
            pltpu.VMEM((8, VOCAB), jnp.float32),
            pltpu.SemaphoreType.DMA,
        ],
    )
    return f(user_ids.astype(jnp.int32), weight)


# speedup vs baseline: 4.7892x; 1.0756x over previous
import jax
import jax.numpy as jnp
from jax import lax
from jax.experimental import pallas as pl
from jax.experimental.pallas import tpu as pltpu
from jax.experimental.pallas import tpu_sc as plsc

NUM_USERS = 100000
VOCAB = 1000
BATCH = 4096
GAIN = 50.0
CLIP = 2000.0

_L = 16
_NW = 32
_BPW = BATCH // _NW
_Q = 4
_NB = _BPW // 8
_NSLICE = VOCAB // _L


def _sc_body(uid_hbm, w_hbm, out_hbm, idx_v, band_v, lane_v, in0, in1,
             out_buf, gs0, gs1):
    wid = lax.axis_index("s") * 2 + lax.axis_index("c")
    base = wid * _BPW
    pltpu.sync_copy(uid_hbm.at[pl.ds(base, _BPW)], idx_v)
    for j in range(_BPW // _L):
        ids = idx_v[pl.ds(j * _L, _L)]
        band_v[pl.ds(j * _L, _L)] = lax.shift_right_logical(ids, 3)
        lane_v[pl.ds(j * _L, _L)] = lax.bitwise_and(ids, 7)

    in_bufs = (in0, in1)
    gsems = (gs0, gs1)

    def fire(q, slot):
        for k in range(_Q):
            row0 = pl.multiple_of(band_v[pl.ds(q * _Q + k, _L)][0] * 8, 8)
            pltpu.async_copy(w_hbm.at[pl.ds(row0, 8)],
                             in_bufs[slot].at[k], gsems[slot])

    def drain(slot):
        for k in range(_Q):
            pltpu.make_async_copy(w_hbm.at[pl.ds(0, 8)],
                                  in_bufs[slot].at[k], gsems[slot]).wait()

    def compute(q, slot, half):
        in_buf = in_bufs[slot]

        def do_row(k, carry):
            lane = lane_v[pl.ds(q * _Q + k, _L)][0]
            for j in range(_NSLICE):
                x = in_buf[k, lane, pl.ds(j * _L, _L)]
                out_buf[half + k, pl.ds(j * _L, _L)] = jnp.clip(
                    x * GAIN, -CLIP, CLIP)
            x = in_buf[k, lane, pl.ds(VOCAB - _L, _L)]
            out_buf[half + k, pl.ds(VOCAB - _L, _L)] = jnp.clip(
                x * GAIN, -CLIP, CLIP)
            return carry

        lax.fori_loop(0, _Q, do_row, 0)

    fire(0, 0)
    fire(1, 1)

    def do_band(s, carry):
        q0 = s * 2

        drain(0)
        compute(q0, 0, 0)

        @pl.when(s < _NB - 1)
        def _():
            fire(q0 + 2, 0)

        drain(1)
        compute(q0 + 1, 1, _Q)

        @pl.when(s < _NB - 1)
        def _():
            fire(q0 + 3, 1)

        pltpu.sync_copy(
            out_buf, out_hbm.at[pl.ds(pl.multiple_of(base + s * 8, 8), 8)])
        return carry

    lax.fori_loop(0, _NB, do_band, 0)


def kernel(user_ids, weight):
    mesh = plsc.VectorSubcoreMesh(core_axis_name="c", subcore_axis_name="s")
    f = pl.kernel(
        _sc_body,
        mesh=mesh,
        out_type=jax.ShapeDtypeStruct((BATCH, VOCAB), jnp.float32),
        scratch_types=[
            pltpu.VMEM((_BPW,), jnp.int32),
            pltpu.VMEM((_BPW + _L,), jnp.int32),
            pltpu.VMEM((_BPW + _L,), jnp.int32),
            pltpu.VMEM((_Q, 8, VOCAB), jnp.float32),
            pltpu.VMEM((_Q, 8, VOCAB), jnp.float32),
            pltpu.VMEM((8, VOCAB), jnp.float32),
            pltpu.SemaphoreType.DMA,
            pltpu.SemaphoreType.DMA,
        ],
    )
    return f(user_ids.astype(jnp.int32), weight)
